# Initial kernel scaffold; baseline (speedup 1.0000x reference)
import jax
import jax.numpy as jnp
from jax.experimental import pallas as pl


def kernel(x, edge_index, W1, b1, W2, b2):
    raise NotImplementedError("write your pallas kernel here")


# speedup vs baseline: 8.4657x; 8.4657x over previous
import functools

import jax
import jax.numpy as jnp
from jax import lax
from jax.experimental import pallas as pl
from jax.experimental.pallas import tpu as pltpu
from jax.experimental.pallas import tpu_sc as plsc

N = 10000
D = 128
E = 320000
NC, NS = 2, 16
NW = NC * NS
CH = 128
EPW = 10240
KCH = EPW // CH
E_PAD = NW * EPW
NP = 10240
RPT = NP // NS
BR = 400

_mesh = plsc.VectorSubcoreMesh(
    core_axis_name="c", subcore_axis_name="s", num_cores=NC, num_subcores=NS)


@functools.partial(
    pl.kernel,
    out_type=jax.ShapeDtypeStruct((NW * NP,), jnp.float32),
    mesh=_mesh,
    compiler_params=pltpu.CompilerParams(needs_layout_passes=False),
    scratch_types=dict(
        hist=pltpu.VMEM((NP,), jnp.float32),
        idx=pltpu.VMEM((KCH, CH), jnp.int32),
    ),
)
def _sc_degree(dst_hbm, z1_hbm, out_hbm, hist, idx):
    c = lax.axis_index("c")
    s = lax.axis_index("s")
    wid = s * NC + c
    pltpu.sync_copy(z1_hbm, hist)
    pltpu.sync_copy(dst_hbm.at[pl.ds(wid * KCH, KCH)], idx)
    onev = jnp.full((16,), 1.0, jnp.float32)

    @pl.loop(0, KCH)
    def _(j):
        for k in range(CH // 16):
            dv = idx[j, pl.ds(k * 16, 16)]
            plsc.addupdate_scatter(hist, [dv], onev)

    pltpu.sync_copy(hist, out_hbm.at[pl.ds(wid * NP, NP)])


@functools.partial(
    pl.kernel,
    out_type=jax.ShapeDtypeStruct((NC * NP, D), jnp.float32),
    mesh=_mesh,
    scratch_types=dict(
        acc=pltpu.VMEM_SHARED((NP, D), jnp.float32),
        sidx=pltpu.VMEM((KCH, CH), jnp.int32),
        didx=pltpu.VMEM((KCH, CH), jnp.int32),
        rows=pltpu.VMEM((CH, D), jnp.float32),
        sem=pltpu.SemaphoreType.DMA,
    ),
)
def _sc_scatter(g_hbm, src_hbm, dst_hbm, z_hbm, out_hbm,
                acc, sidx, didx, rows, sem):
    c = lax.axis_index("c")
    s = lax.axis_index("s")
    wid = s * NC + c
    t0 = s * RPT
    pltpu.sync_copy(z_hbm.at[pl.ds(t0, RPT)], acc.at[pl.ds(t0, RPT)])
    pltpu.sync_copy(src_hbm.at[pl.ds(wid * KCH, KCH)], sidx)
    pltpu.sync_copy(dst_hbm.at[pl.ds(wid * KCH, KCH)], didx)
    plsc.subcore_barrier()

    @pl.loop(0, KCH)
    def _(j):
        pltpu.async_copy(g_hbm.at[sidx.at[j]], rows, sem).wait()
        pltpu.sync_copy(rows, acc.at[didx.at[j]], add=True)

    plsc.subcore_barrier()
    pltpu.sync_copy(acc.at[pl.ds(t0, RPT)], out_hbm.at[pl.ds(c * NP + t0, RPT)])


def _tc_dis_body(degp_ref, o_ref):
    deg = jnp.sum(degp_ref[...], axis=0) + 1.0
    o_ref[...] = lax.rsqrt(deg)


_tc_dis = pl.pallas_call(
    _tc_dis_body, grid=(1,),
    in_specs=[pl.BlockSpec((NW, NP // 128, 128), lambda i: (0, 0, 0))],
    out_specs=pl.BlockSpec((NP // 128, 128), lambda i: (0, 0)),
    out_shape=jax.ShapeDtypeStruct((NP // 128, 128), jnp.float32))


def _tc1_body(dis_ref, x_ref, w_ref, o_ref):
    o_ref[...] = jnp.dot(
        x_ref[...], w_ref[...], preferred_element_type=jnp.float32,
        precision=lax.Precision.HIGHEST) * dis_ref[...]


def _tc2_body(dis_ref, p_ref, g_ref, b_ref, w_ref, o_ref):
    dis = dis_ref[...]
    h = jnp.maximum((p_ref[0] + p_ref[1] + g_ref[...]) * dis + b_ref[...], 0.0)
    o_ref[...] = jnp.dot(
        h, w_ref[...], preferred_element_type=jnp.float32,
        precision=lax.Precision.HIGHEST) * dis


def _tc3_body(dis_ref, p_ref, g_ref, b_ref, o_ref):
    o_ref[...] = ((p_ref[0] + p_ref[1] + g_ref[...]) * dis_ref[...]
                  + b_ref[...])


_dis_spec = pl.BlockSpec((BR, 1), lambda i: (i, 0))
_row_spec = pl.BlockSpec((BR, D), lambda i: (i, 0))
_p_spec = pl.BlockSpec((2, BR, D), lambda i: (0, i, 0))
_w_spec = pl.BlockSpec((D, D), lambda i: (0, 0))
_b_spec = pl.BlockSpec((1, D), lambda i: (0, 0))
_row_out = jax.ShapeDtypeStruct((N, D), jnp.float32)

_tc1 = pl.pallas_call(
    _tc1_body, grid=(N // BR,),
    in_specs=[_dis_spec, _row_spec, _w_spec],
    out_specs=_row_spec, out_shape=_row_out)

_tc2 = pl.pallas_call(
    _tc2_body, grid=(N // BR,),
    in_specs=[_dis_spec, _p_spec, _row_spec, _b_spec, _w_spec],
    out_specs=_row_spec, out_shape=_row_out)

_tc3 = pl.pallas_call(
    _tc3_body, grid=(N // BR,),
    in_specs=[_dis_spec, _p_spec, _row_spec, _b_spec],
    out_specs=_row_spec, out_shape=_row_out)


def kernel(x, edge_index, W1, b1, W2, b2):
    ei = edge_index.astype(jnp.int32)
    pad = E_PAD - E
    src = jnp.concatenate([ei[0], jnp.zeros((pad,), jnp.int32)])
    dst = jnp.concatenate([ei[1], jnp.full((pad,), N, jnp.int32)])
    src = src.reshape(E_PAD // CH, CH)
    dst = dst.reshape(E_PAD // CH, CH)

    z1 = jnp.zeros((NP,), jnp.float32)
    zD = jnp.zeros((NP, D), jnp.float32)
    b1r = b1.reshape(1, D)
    b2r = b2.reshape(1, D)

    degp = _sc_degree(dst, z1).reshape(NW, NP // 128, 128)
    dis = _tc_dis(degp).reshape(NP)[:N].reshape(N, 1)
    g1 = _tc1(dis, x, W1)
    p1 = _sc_scatter(g1, src, dst, zD).reshape(NC, NP, D)
    g2 = _tc2(dis, p1, g1, b1r, W2)
    p2 = _sc_scatter(g2, src, dst, zD).reshape(NC, NP, D)
    out = _tc3(dis, p2, g2, b2r)
    return out
